# baseline (speedup 1.0000x reference)
import functools

import jax
import jax.numpy as jnp
from jax import lax
from jax.experimental import pallas as pl
from jax.experimental.pallas import tpu as pltpu
from jax.experimental.pallas import tpu_sc as plsc

_info = plsc.get_sparse_core_info()
_NC, _NS = _info.num_cores, _info.num_subcores
_NW = _NC * _NS

_CHUNK = 128
_NR = 4
_NO = 2


def _make_gather(B: int, D: int):
    b_per_w = B // _NW
    n_chunks = b_per_w // _CHUNK
    assert n_chunks % _NR == 0

    mesh = plsc.VectorSubcoreMesh(core_axis_name="c", subcore_axis_name="s")

    @functools.partial(
        pl.kernel,
        out_type=jax.ShapeDtypeStruct((B, D), jnp.float32),
        mesh=mesh,
        scratch_types=[
            pltpu.VMEM((n_chunks, _CHUNK), jnp.int32),
            [pltpu.VMEM((_CHUNK, D), jnp.float32) for _ in range(_NR)],
            [pltpu.VMEM_SHARED((_NS * _CHUNK, D), jnp.float32) for _ in range(_NO)],
            [pltpu.SemaphoreType.DMA for _ in range(_NR)],
            [pltpu.SemaphoreType.DMA for _ in range(_NR)],
            [pltpu.SemaphoreType.DMA for _ in range(_NO)],
        ],
    )
    def gather_kernel(
        table_hbm, idx_hbm, out_hbm, idx_v, rows, shared, g_sems, x_sems, o_sems
    ):
        s = lax.axis_index("s")
        wid = s * _NC + lax.axis_index("c")
        out_base = wid * b_per_w

        pltpu.sync_copy(idx_hbm.at[pl.ds(wid * n_chunks, n_chunks)], idx_v)

        def sh(b2):
            return shared[b2].at[pl.ds(s * _CHUNK, _CHUNK)]

        def start_gather(j, b4):
            pltpu.async_copy(table_hbm.at[idx_v.at[j]], rows[b4], g_sems[b4])

        def wait_gather(j, b4):
            pltpu.make_async_copy(
                table_hbm.at[idx_v.at[j]], rows[b4], g_sems[b4]
            ).wait()

        def start_xbar(b4, b2):
            pltpu.async_copy(rows[b4], sh(b2), x_sems[b4])

        def wait_xbar(b4, b2):
            pltpu.make_async_copy(rows[b4], sh(b2), x_sems[b4]).wait()

        def start_out(j, b2):
            pltpu.async_copy(
                sh(b2), out_hbm.at[pl.ds(out_base + j * _CHUNK, _CHUNK)], o_sems[b2]
            )

        def wait_out(b2):
            pltpu.make_async_copy(
                sh(b2), out_hbm.at[pl.ds(out_base, _CHUNK)], o_sems[b2]
            ).wait()

        def step(j, b4, need_wait_out, do_prev, do_gather):
            wait_gather(j, b4)
            if need_wait_out:
                wait_out(b4 % _NO)
            start_xbar(b4, b4 % _NO)
            if do_prev:
                pb4 = (b4 - 1) % _NR
                wait_xbar(pb4, pb4 % _NO)
                start_out(j - 1, pb4 % _NO)
            if do_gather:
                start_gather(j + 2, (b4 + 2) % _NR)

        start_gather(0, 0)
        start_gather(1, 1)
        for b4 in range(_NR):
            step(b4, b4, b4 >= _NO, b4 >= 1, True)

        def body(g, carry):
            for b4 in range(_NR):
                step(g * _NR + b4, b4, True, True, True)
            return carry

        lax.fori_loop(1, n_chunks // _NR - 1, body, 0)

        for b4 in range(_NR):
            step(n_chunks - _NR + b4, b4, True, True, b4 < _NR - 2)

        lb4 = (n_chunks - 1) % _NR
        wait_xbar(lb4, lb4 % _NO)
        start_out(n_chunks - 1, lb4 % _NO)
        for b2 in range(_NO):
            wait_out(b2)

    return gather_kernel


def kernel(item_ids, table):
    ids_shape = item_ids.shape
    B = ids_shape[0] * ids_shape[1]
    D = table.shape[1]
    idx2d = item_ids.reshape(B // _CHUNK, _CHUNK).astype(jnp.int32)
    out = _make_gather(B, D)(table, idx2d)
    return out.reshape(*ids_shape, D)
